# baseline (speedup 1.0000x reference)
import functools

import jax
import jax.numpy as jnp
import numpy as np
from jax import lax
from jax.experimental import pallas as pl
from jax.experimental.pallas import tpu as pltpu
from jax.experimental.pallas import tpu_sc as plsc

_BATCH = 128

_MASK_BITS = (
    "01101011001111001101111010111111011111111111101111111111111111100111"
    "111011111111111111111101111001110010101100001101000111011011"
)
_KEEP_MASK = np.array([b == "1" for b in _MASK_BITS], dtype=bool)
_IDX = np.nonzero(_KEEP_MASK)[0].astype(np.int32)

_GROUP = 8
_COL_CHUNKS = 8
_LANES = 16


def _step_table(idx):
    delta = idx - np.arange(idx.shape[0], dtype=np.int32)
    starts = np.flatnonzero(np.diff(np.concatenate([[0], delta])) != 0)
    steps = np.diff(np.concatenate([[0], delta[starts]]))
    return [(int(s), int(st)) for s, st in zip(starts, steps)]


_STEPS = _step_table(_IDX)


def _src_runs(idx):
    brk = np.flatnonzero(np.diff(idx) != 1)
    starts = np.concatenate([[0], brk + 1])
    ends = np.concatenate([brk, [idx.shape[0] - 1]])
    return [(int(idx[s]), int(idx[e] - idx[s] + 1)) for s, e in zip(starts, ends)]


_RUNS = _src_runs(_IDX)


@functools.cache
def _gather_fn(keep_size, d_model, num_cores, num_subcores):
    mesh = plsc.VectorSubcoreMesh(core_axis_name="c", subcore_axis_name="s")
    nw = num_cores * num_subcores
    n_groups = keep_size // _GROUP
    chunk = d_model // _COL_CHUNKS
    tasks_per_w = n_groups * _COL_CHUNKS // nw
    groups_per_band = nw // _COL_CHUNKS

    @functools.partial(
        pl.kernel,
        mesh=mesh,
        out_type=jax.ShapeDtypeStruct((keep_size, d_model), jnp.float32),
        scratch_types=[
            pltpu.VMEM((keep_size,), jnp.int32),
        ]
        + [pltpu.VMEM((_GROUP, chunk), jnp.float32) for _ in range(tasks_per_w)]
        + [pltpu.SemaphoreType.DMA, pltpu.SemaphoreType.DMA],
    )
    def k(x_hbm, out_hbm, sidx_v, *bufs_and_sems):
        bufs = bufs_and_sems[:tasks_per_w]
        sem_g, sem_s = bufs_and_sems[tasks_per_w:]
        wid = lax.axis_index("s") * num_cores + lax.axis_index("c")
        for t in range(keep_size // _LANES):
            dst = lax.iota(jnp.int32, _LANES) + (_LANES * t)
            src = dst
            for start, step in _STEPS:
                src = src + jnp.where(dst >= start, jnp.int32(step), jnp.int32(0))
            sidx_v[pl.ds(_LANES * t, _LANES)] = src
        gathers = []
        rows = []
        cols = []
        for t in range(tasks_per_w):
            g = wid // _COL_CHUNKS + t * groups_per_band
            row = pl.multiple_of(g * _GROUP, _GROUP)
            col = pl.multiple_of(((wid + t) % _COL_CHUNKS) * chunk, chunk)
            rows.append(row)
            cols.append(col)
            gathers.append(
                pltpu.async_copy(
                    x_hbm.at[sidx_v.at[pl.ds(row, _GROUP)], pl.ds(col, chunk)],
                    bufs[t],
                    sem_g,
                )
            )
        writes = []
        for t in range(tasks_per_w):
            gathers[t].wait()
            writes.append(
                pltpu.async_copy(
                    bufs[t],
                    out_hbm.at[pl.ds(rows[t], _GROUP), pl.ds(cols[t], chunk)],
                    sem_s,
                )
            )
        for w in writes:
            w.wait()

    return k


def kernel(inputs):
    batch_size, d_model = inputs.shape
    assert batch_size == _BATCH, "shapes are fixed by the problem definition"
    keep_size = int(_IDX.shape[0])

    info = plsc.get_sparse_core_info()
    fn = _gather_fn(keep_size, d_model, info.num_cores, info.num_subcores)
    out = fn(inputs)
    row = jnp.arange(batch_size, dtype=jnp.int32)
    kept = jnp.zeros((batch_size,), dtype=jnp.bool_)
    for s, l in _RUNS:
        kept = kept | ((row >= s) & (row < s + l))
    return out, kept


# speedup vs baseline: 1.0144x; 1.0144x over previous
import functools

import jax
import jax.numpy as jnp
import numpy as np
from jax import lax
from jax.experimental import pallas as pl
from jax.experimental.pallas import tpu as pltpu
from jax.experimental.pallas import tpu_sc as plsc

_BATCH = 128

_MASK_BITS = (
    "01101011001111001101111010111111011111111111101111111111111111100111"
    "111011111111111111111101111001110010101100001101000111011011"
)
_KEEP_MASK = np.array([b == "1" for b in _MASK_BITS], dtype=bool)
_IDX = np.nonzero(_KEEP_MASK)[0].astype(np.int32)

_GROUP = 8
_COL_CHUNKS = 8
_LANES = 16


def _step_table(idx):
    delta = idx - np.arange(idx.shape[0], dtype=np.int32)
    starts = np.flatnonzero(np.diff(np.concatenate([[0], delta])) != 0)
    steps = np.diff(np.concatenate([[0], delta[starts]]))
    return [(int(s), int(st)) for s, st in zip(starts, steps)]


_STEPS = _step_table(_IDX)


def _src_runs(idx):
    brk = np.flatnonzero(np.diff(idx) != 1)
    starts = np.concatenate([[0], brk + 1])
    ends = np.concatenate([brk, [idx.shape[0] - 1]])
    return [(int(idx[s]), int(idx[e] - idx[s] + 1)) for s, e in zip(starts, ends)]


_RUNS = _src_runs(_IDX)


@functools.cache
def _gather_fn(keep_size, d_model, num_cores, num_subcores):
    mesh = plsc.VectorSubcoreMesh(core_axis_name="c", subcore_axis_name="s")
    nw = num_cores * num_subcores
    n_groups = keep_size // _GROUP
    chunk = d_model // _COL_CHUNKS
    tasks_per_w = n_groups * _COL_CHUNKS // nw
    groups_per_band = nw // _COL_CHUNKS

    @functools.partial(
        pl.kernel,
        mesh=mesh,
        out_type=jax.ShapeDtypeStruct((keep_size, d_model), jnp.float32),
        scratch_types=[
            pltpu.VMEM((keep_size,), jnp.int32),
        ]
        + [pltpu.VMEM((_GROUP, chunk), jnp.float32) for _ in range(tasks_per_w)]
        + [pltpu.SemaphoreType.DMA, pltpu.SemaphoreType.DMA],
    )
    def k(x_hbm, out_hbm, sidx_v, *bufs_and_sems):
        bufs = bufs_and_sems[:tasks_per_w]
        sem_g, sem_s = bufs_and_sems[tasks_per_w:]
        wid = lax.axis_index("s") * num_cores + lax.axis_index("c")
        gathers = []
        rows = []
        cols = []
        for t in range(tasks_per_w):
            g = wid // _COL_CHUNKS + t * groups_per_band
            row = pl.multiple_of(g * _GROUP, _GROUP)
            col = pl.multiple_of(((wid + t) % _COL_CHUNKS) * chunk, chunk)
            rows.append(row)
            cols.append(col)
            voff = pl.multiple_of((row // _LANES) * _LANES, _LANES)
            dst = lax.iota(jnp.int32, _LANES) + voff
            src = dst
            for start, step in _STEPS:
                src = src + jnp.where(dst >= start, jnp.int32(step), jnp.int32(0))
            sidx_v[pl.ds(voff, _LANES)] = src
            gathers.append(
                pltpu.async_copy(
                    x_hbm.at[sidx_v.at[pl.ds(row, _GROUP)], pl.ds(col, chunk)],
                    bufs[t],
                    sem_g,
                )
            )
        writes = []
        for t in range(tasks_per_w):
            gathers[t].wait()
            writes.append(
                pltpu.async_copy(
                    bufs[t],
                    out_hbm.at[pl.ds(rows[t], _GROUP), pl.ds(cols[t], chunk)],
                    sem_s,
                )
            )
        for w in writes:
            w.wait()

    return k


def kernel(inputs):
    batch_size, d_model = inputs.shape
    assert batch_size == _BATCH, "shapes are fixed by the problem definition"
    keep_size = int(_IDX.shape[0])

    info = plsc.get_sparse_core_info()
    fn = _gather_fn(keep_size, d_model, info.num_cores, info.num_subcores)
    out = fn(inputs)
    row = jnp.arange(batch_size, dtype=jnp.int32)
    kept = jnp.zeros((batch_size,), dtype=jnp.bool_)
    for s, l in _RUNS:
        kept = kept | ((row >= s) & (row < s + l))
    return out, kept
